# baseline (speedup 1.0000x reference)
import jax
import jax.numpy as jnp
from jax.experimental import pallas as pl
from jax.experimental.pallas import tpu as pltpu

_BN = 2048


def _fused_kernel(mask_ref, x_ref, pt_ref, lh_ref, loss_ref, hm_ref,
                  acc_ref, hist_ref, ms_ref):
    i = pl.program_id(0)
    nsteps = pl.num_programs(0)
    x = x_ref[...]
    m = mask_ref[...]
    C = x.shape[0]

    BN = x.shape[1]
    colmax = jnp.max(x, axis=0, keepdims=True)
    d = x - colmax
    e = jnp.exp(d).astype(jnp.bfloat16)
    ones_r = jnp.ones((16, C), jnp.bfloat16)
    denom16 = jax.lax.dot_general(ones_r, e, (((1,), (0,)), ((), ())),
                                  preferred_element_type=jnp.float32)
    denom = denom16[0:1, :]
    w = (m / denom).astype(jnp.bfloat16)
    w16 = jnp.broadcast_to(w, (16, BN))
    contrib16 = jax.lax.dot_general(e, w16, (((1,), (1,)), ((), ())),
                                    preferred_element_type=jnp.float32)
    contrib = contrib16[:, 0:1]

    onehot = jnp.where(d == 0.0, jnp.float32(1), jnp.float32(0))
    hcontrib = jax.lax.dot_general(onehot, m, (((1,), (1,)), ((), ())),
                                   preferred_element_type=jnp.float32)

    @pl.when(i == 0)
    def _():
        acc_ref[...] = jnp.zeros_like(acc_ref)
        hist_ref[...] = jnp.zeros_like(hist_ref)
        ms_ref[0] = 0.0

    acc_ref[...] += contrib
    hist_ref[...] += hcontrib
    ms_ref[0] += jnp.sum(m)

    @pl.when(i == nsteps - 1)
    def _():
        hist = hist_ref[...]
        s = ms_ref[0]
        histogram = hist / s
        mean_probs = acc_ref[...] / s
        pt_c = pt_ref[...].reshape(C, 1)
        lh_c = lh_ref[...].reshape(C, 1)
        inv_lh = 1.0 / lh_c
        sc_pt = jnp.where(jnp.isinf(inv_lh), 0.0, inv_lh)
        mod_pt = pt_c * sc_pt
        mod_pt = mod_pt / jnp.sum(mod_pt)
        inv_h = 1.0 / histogram
        sc_ps = jnp.where(jnp.isinf(inv_h), 0.0, inv_h)
        mod_ps = mean_probs * sc_ps
        mod_ps = mod_ps / jnp.sum(mod_ps)
        loss = jnp.sum(mod_pt * jnp.log(mod_ps + 1e-9))
        loss_ref[...] = loss.reshape(1, 1)
        hm_ref[...] = jnp.mean(histogram).reshape(1, 1)


def kernel(mask, logits_ulb_s, p_t, label_hist):
    B, C = logits_ulb_s.shape
    bn = _BN
    grid = B // bn
    dt = logits_ulb_s.dtype
    xt = logits_ulb_s.T
    mask_f = mask.astype(dt).reshape(1, B)
    pt2 = p_t.reshape(1, C)
    lh2 = label_hist.reshape(1, C)
    loss, hm = pl.pallas_call(
        _fused_kernel,
        grid=(grid,),
        in_specs=[
            pl.BlockSpec((1, bn), lambda i: (0, i)),
            pl.BlockSpec((C, bn), lambda i: (0, i)),
            pl.BlockSpec((1, C), lambda i: (0, 0)),
            pl.BlockSpec((1, C), lambda i: (0, 0)),
        ],
        out_specs=[
            pl.BlockSpec((1, 1), lambda i: (0, 0)),
            pl.BlockSpec((1, 1), lambda i: (0, 0)),
        ],
        out_shape=[
            jax.ShapeDtypeStruct((1, 1), dt),
            jax.ShapeDtypeStruct((1, 1), dt),
        ],
        scratch_shapes=[
            pltpu.VMEM((C, 1), dt),
            pltpu.VMEM((C, 1), dt),
            pltpu.SMEM((1,), dt),
        ],
    )(mask_f, xt, pt2, lh2)
    return loss[0, 0], hm[0, 0]


# speedup vs baseline: 1.0569x; 1.0569x over previous
import jax
import jax.numpy as jnp
from jax.experimental import pallas as pl
from jax.experimental.pallas import tpu as pltpu

_BN = 1024


def _fused_kernel(mask_ref, x_ref, pt_ref, lh_ref, loss_ref, hm_ref,
                  acc_ref, hist_ref, ms_ref):
    i = pl.program_id(0)
    nsteps = pl.num_programs(0)
    x = x_ref[...]
    m = mask_ref[...]
    C = x.shape[0]

    colmax = jnp.max(x, axis=0, keepdims=True)
    e = jnp.exp(x - colmax)
    ones_r = jnp.ones((1, C), x.dtype)
    denom = jax.lax.dot_general(ones_r, e, (((1,), (0,)), ((), ())),
                                preferred_element_type=jnp.float32)
    w = m / denom
    contrib = jax.lax.dot_general(e, w, (((1,), (1,)), ((), ())),
                                  preferred_element_type=jnp.float32)

    fe = jnp.floor(e)
    hcontrib = jax.lax.dot_general(fe, m, (((1,), (1,)), ((), ())),
                                   preferred_element_type=jnp.float32)

    @pl.when(i == 0)
    def _():
        acc_ref[...] = jnp.zeros_like(acc_ref)
        hist_ref[...] = jnp.zeros_like(hist_ref)
        ms_ref[0] = 0.0

    acc_ref[...] += contrib
    hist_ref[...] += hcontrib
    ms_ref[0] += jnp.sum(m)

    @pl.when(i == nsteps - 1)
    def _():
        hist = hist_ref[...]
        s = ms_ref[0]
        histogram = hist / s
        mean_probs = acc_ref[...] / s
        pt_c = pt_ref[...].reshape(C, 1)
        lh_c = lh_ref[...].reshape(C, 1)
        inv_lh = 1.0 / lh_c
        sc_pt = jnp.where(jnp.isinf(inv_lh), 0.0, inv_lh)
        mod_pt = pt_c * sc_pt
        mod_pt = mod_pt / jnp.sum(mod_pt)
        inv_h = 1.0 / histogram
        sc_ps = jnp.where(jnp.isinf(inv_h), 0.0, inv_h)
        mod_ps = mean_probs * sc_ps
        mod_ps = mod_ps / jnp.sum(mod_ps)
        loss = jnp.sum(mod_pt * jnp.log(mod_ps + 1e-9))
        loss_ref[...] = loss.reshape(1, 1)
        hm_ref[...] = jnp.mean(histogram).reshape(1, 1)


def kernel(mask, logits_ulb_s, p_t, label_hist):
    B, C = logits_ulb_s.shape
    bn = _BN
    grid = B // bn
    dt = logits_ulb_s.dtype
    xt = logits_ulb_s.T
    mask_f = mask.astype(dt).reshape(1, B)
    pt2 = p_t.reshape(1, C)
    lh2 = label_hist.reshape(1, C)
    loss, hm = pl.pallas_call(
        _fused_kernel,
        grid=(grid,),
        in_specs=[
            pl.BlockSpec((1, bn), lambda i: (0, i)),
            pl.BlockSpec((C, bn), lambda i: (0, i)),
            pl.BlockSpec((1, C), lambda i: (0, 0)),
            pl.BlockSpec((1, C), lambda i: (0, 0)),
        ],
        out_specs=[
            pl.BlockSpec((1, 1), lambda i: (0, 0)),
            pl.BlockSpec((1, 1), lambda i: (0, 0)),
        ],
        out_shape=[
            jax.ShapeDtypeStruct((1, 1), dt),
            jax.ShapeDtypeStruct((1, 1), dt),
        ],
        scratch_shapes=[
            pltpu.VMEM((C, 1), dt),
            pltpu.VMEM((C, 1), dt),
            pltpu.SMEM((1,), dt),
        ],
    )(mask_f, xt, pt2, lh2)
    return loss[0, 0], hm[0, 0]


# speedup vs baseline: 1.1600x; 1.0976x over previous
import jax
import jax.numpy as jnp
from jax.experimental import pallas as pl
from jax.experimental.pallas import tpu as pltpu

_BN = 2048


def _fused_kernel(mask_ref, x_ref, pt_ref, lh_ref, loss_ref, hm_ref,
                  acc_ref, hist_ref, ms_ref):
    i = pl.program_id(0)
    nsteps = pl.num_programs(0)
    x = x_ref[...]
    m = mask_ref[...]
    C = x.shape[0]

    colmax = jnp.max(x, axis=0, keepdims=True)
    e = jnp.exp(x - colmax)
    ones_r = jnp.ones((1, C), x.dtype)
    denom = jax.lax.dot_general(ones_r, e, (((1,), (0,)), ((), ())),
                                preferred_element_type=jnp.float32)
    w = m / denom
    contrib = jax.lax.dot_general(e, w, (((1,), (1,)), ((), ())),
                                  preferred_element_type=jnp.float32)

    fe = jnp.floor(e)
    hcontrib = jax.lax.dot_general(fe, m, (((1,), (1,)), ((), ())),
                                   preferred_element_type=jnp.float32)

    @pl.when(i == 0)
    def _():
        acc_ref[...] = jnp.zeros_like(acc_ref)
        hist_ref[...] = jnp.zeros_like(hist_ref)
        ms_ref[0] = 0.0

    acc_ref[...] += contrib
    hist_ref[...] += hcontrib
    ms_ref[0] += jnp.sum(m)

    @pl.when(i == nsteps - 1)
    def _():
        hist = hist_ref[...]
        s = ms_ref[0]
        histogram = hist / s
        mean_probs = acc_ref[...] / s
        pt_c = pt_ref[...].reshape(C, 1)
        lh_c = lh_ref[...].reshape(C, 1)
        inv_lh = 1.0 / lh_c
        sc_pt = jnp.where(jnp.isinf(inv_lh), 0.0, inv_lh)
        mod_pt = pt_c * sc_pt
        mod_pt = mod_pt / jnp.sum(mod_pt)
        inv_h = 1.0 / histogram
        sc_ps = jnp.where(jnp.isinf(inv_h), 0.0, inv_h)
        mod_ps = mean_probs * sc_ps
        mod_ps = mod_ps / jnp.sum(mod_ps)
        loss = jnp.sum(mod_pt * jnp.log(mod_ps + 1e-9))
        loss_ref[...] = loss.reshape(1, 1)
        hm_ref[...] = jnp.mean(histogram).reshape(1, 1)


def kernel(mask, logits_ulb_s, p_t, label_hist):
    B, C = logits_ulb_s.shape
    bn = _BN
    grid = B // bn
    dt = logits_ulb_s.dtype
    xt = logits_ulb_s.T
    mask_f = mask.astype(dt).reshape(1, B)
    pt2 = p_t.reshape(1, C)
    lh2 = label_hist.reshape(1, C)
    loss, hm = pl.pallas_call(
        _fused_kernel,
        grid=(grid,),
        in_specs=[
            pl.BlockSpec((1, bn), lambda i: (0, i)),
            pl.BlockSpec((C, bn), lambda i: (0, i)),
            pl.BlockSpec((1, C), lambda i: (0, 0)),
            pl.BlockSpec((1, C), lambda i: (0, 0)),
        ],
        out_specs=[
            pl.BlockSpec((1, 1), lambda i: (0, 0)),
            pl.BlockSpec((1, 1), lambda i: (0, 0)),
        ],
        out_shape=[
            jax.ShapeDtypeStruct((1, 1), dt),
            jax.ShapeDtypeStruct((1, 1), dt),
        ],
        scratch_shapes=[
            pltpu.VMEM((C, 1), dt),
            pltpu.VMEM((C, 1), dt),
            pltpu.SMEM((1,), dt),
        ],
    )(mask_f, xt, pt2, lh2)
    return loss[0, 0], hm[0, 0]
